# baseline (device time: 561670 ns/iter reference)
import functools

import jax
import jax.numpy as jnp
from jax import lax
from jax.experimental import pallas as pl
from jax.experimental.pallas import tpu as pltpu

Y = 4
QW = 256
M_PER = 4096
N_PER = 1024

UNITS = [
    (1, 0, 128), (1, 128, 128),
    (2, 0, QW), (3, 0, QW),
]
NU = len(UNITS)


def kernel(x):
    def body(x_ref, out_ref, local_sem,
             ysend, yrecv, xsend, xrecv, zsend, zrecv):
        my_x = lax.axis_index("x")
        my_y = lax.axis_index("y")
        my_z = lax.axis_index("z")
        a = lax.rem(my_z + 2 * my_x, 4)
        b = lax.rem(a + 2, 4)
        zp = my_z ^ 1

        def sig(sem, dev):
            pl.semaphore_signal(
                sem, inc=1, device_id=dev,
                device_id_type=pl.DeviceIdType.MESH,
            )

        def barrier_with_partners(sem):
            for d in range(1, Y):
                q = lax.rem(my_y + d, Y)
                sig(sem, (my_x, q, my_z))
            sig(sem, (1 - my_x, my_y, my_z))
            sig(sem, (my_x, my_y, zp))
            pl.semaphore_wait(sem, 5)

        barrier_with_partners(pltpu.get_barrier_semaphore())

        local = pltpu.make_async_copy(
            x_ref.at[:, pl.ds(my_y * N_PER, N_PER)],
            out_ref.at[pl.ds(my_y * M_PER, M_PER), :],
            local_sem,
        )
        local.start()

        def block_row(d):
            return lax.rem(my_y - d + Y, Y) * M_PER

        yrd = []
        for slot, (d, co, w) in enumerate(UNITS):
            qdev = lax.rem(my_y + d, Y)
            r = pltpu.make_async_remote_copy(
                src_ref=x_ref.at[:, pl.ds(qdev * N_PER + a * QW + co, w)],
                dst_ref=out_ref.at[pl.ds(my_y * M_PER, M_PER),
                                   pl.ds(a * QW + co, w)],
                send_sem=ysend.at[slot],
                recv_sem=yrecv.at[slot],
                device_id=(my_x, qdev, my_z),
                device_id_type=pl.DeviceIdType.MESH,
            )
            r.start()
            yrd.append(r)

        def unit_copy(slot, quarter, send_ref, recv_ref, dev):
            d, co, w = UNITS[slot]
            p_row = block_row(d)
            return pltpu.make_async_remote_copy(
                src_ref=out_ref.at[pl.ds(p_row, M_PER),
                                   pl.ds(quarter * QW + co, w)],
                dst_ref=out_ref.at[pl.ds(p_row, M_PER),
                                   pl.ds(quarter * QW + co, w)],
                send_sem=send_ref,
                recv_sem=recv_ref,
                device_id=dev,
                device_id_type=pl.DeviceIdType.MESH,
            )

        xrd = []
        zrd = []
        for slot in range(NU):
            yrd[slot].wait_recv()
            r = unit_copy(slot, a, xsend.at[slot], xrecv.at[slot],
                          (1 - my_x, my_y, my_z))
            r.start()
            xrd.append(r)
            r = unit_copy(slot, a, zsend.at[slot], zrecv.at[slot],
                          (my_x, my_y, zp))
            r.start()
            zrd.append(r)

        for slot in range(NU):
            xrd[slot].wait_recv()
            r = unit_copy(slot, b, zsend.at[NU + slot],
                          zrecv.at[NU + slot], (my_x, my_y, zp))
            r.start()
            zrd.append(r)

        for s in range(2 * NU):
            r = unit_copy(s % NU, 0, zsend.at[0], zrecv.at[s],
                          (my_x, my_y, my_z))
            r.wait_recv()

        for r in yrd:
            r.wait_send()
        for r in xrd:
            r.wait_send()
        for r in zrd:
            r.wait_send()

        local.wait()

        @functools.partial(
            pl.run_scoped, exit_sem=pltpu.SemaphoreType.REGULAR
        )
        def _(exit_sem):
            barrier_with_partners(exit_sem)

    return pl.pallas_call(
        body,
        out_shape=jax.ShapeDtypeStruct((Y * M_PER, N_PER), x.dtype),
        in_specs=[pl.BlockSpec(memory_space=pltpu.MemorySpace.HBM)],
        out_specs=pl.BlockSpec(memory_space=pltpu.MemorySpace.HBM),
        scratch_shapes=[
            pltpu.SemaphoreType.DMA,
            pltpu.SemaphoreType.DMA((NU,)),
            pltpu.SemaphoreType.DMA((NU,)),
            pltpu.SemaphoreType.DMA((NU,)),
            pltpu.SemaphoreType.DMA((NU,)),
            pltpu.SemaphoreType.DMA((2 * NU,)),
            pltpu.SemaphoreType.DMA((2 * NU,)),
        ],
        compiler_params=pltpu.CompilerParams(collective_id=0),
    )(x)
